# baseline (speedup 1.0000x reference)
import functools

import jax
import jax.numpy as jnp
from jax import lax
from jax.experimental import pallas as pl
from jax.experimental.pallas import tpu as pltpu
from jax.experimental.pallas import tpu_sc as plsc

Q = 1024
K_ORIG = 100000
NCHUNK = 784
K_PAD = NCHUNK * 128
QT = 32
GRID = Q // QT

def _argmin_body(x_ref, in_ref, out_ref):
    xt = x_ref[0]

    def step(c, carry):
        best, bidx = carry
        chunk = in_ref[c, :]
        d = jnp.abs(chunk[None, :] - xt)
        m = d < best
        return jnp.where(m, d, best), jnp.where(m, c, bidx)

    best0 = jnp.full((QT, 128), jnp.inf, jnp.float32)
    bidx0 = jnp.zeros((QT, 128), jnp.int32)
    best, bidx = lax.fori_loop(0, NCHUNK, step, (best0, bidx0))

    lane = lax.broadcasted_iota(jnp.int32, (QT, 128), 1)
    kfull = bidx * 128 + lane
    mv = jnp.min(best, axis=1, keepdims=True)
    kc = jnp.where(best == mv, kfull, 1 << 30)
    out_ref[0, 0, :] = jnp.min(kc, axis=1)


def _tc_argmin(x, inp):
    xr = x.reshape(GRID, QT, 1)
    ir = inp.reshape(NCHUNK, 128)
    out = pl.pallas_call(
        _argmin_body,
        grid=(GRID,),
        in_specs=[
            pl.BlockSpec((1, QT, 1), lambda i: (i, 0, 0)),
            pl.BlockSpec((NCHUNK, 128), lambda i: (0, 0)),
        ],
        out_specs=pl.BlockSpec((1, 1, QT), lambda i: (i, 0, 0)),
        out_shape=jax.ShapeDtypeStruct((GRID, 1, QT), jnp.int32),
    )(xr, ir)
    return out.reshape(Q)


_NW = 32
_PER = Q // _NW


def _sc_gather(acc, idx):
    mesh = plsc.VectorSubcoreMesh(core_axis_name="c", subcore_axis_name="s")

    @functools.partial(
        pl.kernel, mesh=mesh,
        out_type=jax.ShapeDtypeStruct((Q,), jnp.float32),
        scratch_types=[
            pltpu.VMEM((_PER,), jnp.int32),
            pltpu.VMEM((_PER,), jnp.float32),
            pltpu.SemaphoreType.DMA,
        ],
    )
    def body(acc_hbm, idx_hbm, out_hbm, idx_v, vals_v, sem):
        wid = lax.axis_index("s") * 2 + lax.axis_index("c")
        base = wid * _PER
        pltpu.sync_copy(idx_hbm.at[pl.ds(base, _PER)], idx_v)
        pltpu.async_copy(acc_hbm.at[idx_v], vals_v, sem).wait()
        pltpu.sync_copy(vals_v, out_hbm.at[pl.ds(base, _PER)])

    return body(acc, idx)


def kernel(x, input_tensor, accuracy_tensor):
    inp = jnp.pad(input_tensor, (0, K_PAD - K_ORIG), constant_values=jnp.inf)
    idx = _tc_argmin(x, inp)
    return _sc_gather(accuracy_tensor, idx)


# speedup vs baseline: 14.0037x; 14.0037x over previous
import functools

import jax
import jax.numpy as jnp
from jax import lax
from jax.experimental import pallas as pl
from jax.experimental.pallas import tpu as pltpu
from jax.experimental.pallas import tpu_sc as plsc

Q = 1024
K_ORIG = 100000
NT = 32
PER = 3136
K_PAD = NT * PER
NB = 4096
QV = Q // 16

_mesh = plsc.VectorSubcoreMesh(core_axis_name="c", subcore_axis_name="s")
_params = pltpu.CompilerParams(needs_layout_passes=False)


def _build_search_body(x_hbm, keys_hbm, outd_hbm, outi_hbm,
                       keys_v, xq_v, sv_v, si_v, off_v, ctr_v, claim_v,
                       bd_v, bi_v):
    wid = lax.axis_index("s") * 2 + lax.axis_index("c")
    base = wid * PER
    iota = lax.iota(jnp.int32, 16)
    ones_i = jnp.ones((16,), jnp.int32)

    pltpu.sync_copy(keys_hbm.at[pl.ds(base, PER)], keys_v)
    pltpu.sync_copy(x_hbm, xq_v)

    def mm_step(i, c):
        mn, mx = c
        k = keys_v[pl.ds(i * 16, 16)]
        mn = jnp.minimum(mn, k)
        mx = jnp.maximum(mx, jnp.where(k == jnp.inf, -jnp.inf, k))
        return mn, mx

    mn0 = jnp.full((16,), jnp.inf, jnp.float32)
    mx0 = jnp.full((16,), -jnp.inf, jnp.float32)
    mn, mx = lax.fori_loop(0, PER // 16, mm_step, (mn0, mx0))
    vmn = jnp.min(mn)
    vmx = jnp.max(mx)
    width = jnp.maximum(vmx - vmn, jnp.float32(1e-30))
    w16 = jnp.broadcast_to(width, (16,))
    ebits = (lax.bitcast_convert_type(w16, jnp.int32) >> 23) & 0xFF
    scale = lax.bitcast_convert_type((265 - ebits) << 23, jnp.float32)
    invb = lax.bitcast_convert_type((ebits - 11) << 23, jnp.float32)

    def kbin(v):
        return jnp.clip((v - vmn) * scale, 0.0, float(NB - 1)).astype(jnp.int32)

    def z_step(i, _):
        ctr_v[pl.ds(i * 16, 16)] = jnp.zeros((16,), jnp.int32)
        return 0

    lax.fori_loop(0, NB // 16, z_step, 0)

    def h_step(i, _):
        k = keys_v[pl.ds(i * 16, 16)]
        plsc.addupdate_scatter(ctr_v, [kbin(k)], ones_i)
        return 0

    lax.fori_loop(0, PER // 16, h_step, 0)

    def s_step(i, carry):
        cnt = ctr_v[pl.ds(i * 16, 16)]
        cs = plsc.cumsum(cnt)
        excl = cs - cnt + carry
        off_v[pl.ds(i * 16, 16)] = excl
        ctr_v[pl.ds(i * 16, 16)] = excl
        return carry + jnp.sum(cnt)

    lax.fori_loop(0, NB // 16, s_step, jnp.int32(0))

    def p_step(i, _):
        k = keys_v[pl.ds(i * 16, 16)]
        g = base + i * 16 + iota
        b = kbin(k)

        def cond(c):
            return jnp.any(c[0])

        def body(c):
            act = c[0]
            plsc.store_scatter(claim_v, [b], iota, mask=act)
            win = (plsc.load_gather(claim_v, [b]) == iota) & act
            pos = plsc.load_gather(ctr_v, [b])
            plsc.store_scatter(sv_v, [pos], k, mask=win)
            plsc.store_scatter(si_v, [pos], g, mask=win)
            plsc.addupdate_scatter(ctr_v, [b], ones_i, mask=win)
            return (act & jnp.logical_not(win),)

        lax.while_loop(cond, body, (jnp.ones((16,), jnp.bool_),))
        return 0

    lax.fori_loop(0, PER // 16, p_step, 0)

    def q_step(j, _):
        q = xq_v[pl.ds(j * 16, 16)]
        cq = kbin(q)
        p0 = plsc.load_gather(off_v, [cq])
        bd0 = jnp.full((16,), jnp.inf, jnp.float32)
        bi0 = jnp.full((16,), 1 << 30, jnp.int32)

        def upd_best(act, v, gi, bd, bi):
            d = jnp.abs(q - v)
            u = act & ((d < bd) | ((d == bd) & (gi < bi)))
            return jnp.where(u, d, bd), jnp.where(u, gi, bi), d

        def up_cond(c):
            return jnp.any(c[0])

        def up_body(c):
            act, cur, bd, bi = c
            cc = jnp.clip(cur, 0, PER - 1)
            v = plsc.load_gather(sv_v, [cc])
            gi = plsc.load_gather(si_v, [cc])
            bd, bi, _ = upd_best(act, v, gi, bd, bi)
            cb = kbin(v)
            lowb = vmn + (cb - 1).astype(jnp.float32) * invb
            stop = (lowb - q) > bd
            cur = cur + 1
            act = act & jnp.logical_not(stop) & (cur < PER)
            return act, cur, bd, bi

        act0 = p0 < PER
        _, _, bd, bi = lax.while_loop(up_cond, up_body, (act0, p0, bd0, bi0))

        def dn_cond(c):
            return jnp.any(c[0])

        def dn_body(c):
            act, cur, bd, bi = c
            cc = jnp.clip(cur, 0, PER - 1)
            v = plsc.load_gather(sv_v, [cc])
            gi = plsc.load_gather(si_v, [cc])
            bd, bi, _ = upd_best(act, v, gi, bd, bi)
            cb = kbin(v)
            upb = vmn + (cb + 2).astype(jnp.float32) * invb
            stop = (q - upb) > bd
            cur = cur - 1
            act = act & jnp.logical_not(stop) & (cur >= 0)
            return act, cur, bd, bi

        actd0 = (p0 - 1) >= 0
        _, _, bd, bi = lax.while_loop(dn_cond, dn_body, (actd0, p0 - 1, bd, bi))

        bd_v[pl.ds(j * 16, 16)] = bd
        bi_v[pl.ds(j * 16, 16)] = bi
        return 0

    lax.fori_loop(0, QV, q_step, 0)

    pltpu.sync_copy(bd_v, outd_hbm.at[wid])
    pltpu.sync_copy(bi_v, outi_hbm.at[wid])


@functools.partial(
    pl.kernel, mesh=_mesh, compiler_params=_params,
    out_type=[jax.ShapeDtypeStruct((NT, Q), jnp.float32),
              jax.ShapeDtypeStruct((NT, Q), jnp.int32)],
    scratch_types=[
        pltpu.VMEM((PER,), jnp.float32),
        pltpu.VMEM((Q,), jnp.float32),
        pltpu.VMEM((PER,), jnp.float32),
        pltpu.VMEM((PER,), jnp.int32),
        pltpu.VMEM((NB,), jnp.int32),
        pltpu.VMEM((NB,), jnp.int32),
        pltpu.VMEM((NB,), jnp.int32),
        pltpu.VMEM((Q,), jnp.float32),
        pltpu.VMEM((Q,), jnp.int32),
    ],
)
def _build_search(x_hbm, keys_hbm, outd_hbm, outi_hbm,
                  keys_v, xq_v, sv_v, si_v, off_v, ctr_v, claim_v, bd_v, bi_v):
    _build_search_body(x_hbm, keys_hbm, outd_hbm, outi_hbm,
                       keys_v, xq_v, sv_v, si_v, off_v, ctr_v, claim_v,
                       bd_v, bi_v)


@functools.partial(
    pl.kernel, mesh=_mesh, compiler_params=_params,
    out_type=jax.ShapeDtypeStruct((Q,), jnp.float32),
    scratch_types=[
        pltpu.VMEM((32,), jnp.float32),
        pltpu.VMEM((32,), jnp.int32),
        pltpu.VMEM((32,), jnp.int32),
        pltpu.VMEM((32,), jnp.float32),
        pltpu.SemaphoreType.DMA,
    ],
)
def _merge_gather(d_hbm, i_hbm, acc_hbm, out_hbm, dbuf, ibuf, win_i, vbuf, sem):
    wid = lax.axis_index("s") * 2 + lax.axis_index("c")
    qbase = wid * 32

    def m_step(t, c):
        bd0, bi0, bd1, bi1 = c
        pltpu.sync_copy(d_hbm.at[t, pl.ds(qbase, 32)], dbuf)
        pltpu.sync_copy(i_hbm.at[t, pl.ds(qbase, 32)], ibuf)

        def mix(bd, bi, d, gi):
            u = (d < bd) | ((d == bd) & (gi < bi))
            return jnp.where(u, d, bd), jnp.where(u, gi, bi)

        bd0, bi0 = mix(bd0, bi0, dbuf[pl.ds(0, 16)], ibuf[pl.ds(0, 16)])
        bd1, bi1 = mix(bd1, bi1, dbuf[pl.ds(16, 16)], ibuf[pl.ds(16, 16)])
        return bd0, bi0, bd1, bi1

    inf_v = jnp.full((16,), jnp.inf, jnp.float32)
    big_v = jnp.full((16,), 1 << 30, jnp.int32)
    _, bi0, _, bi1 = lax.fori_loop(0, NT, m_step, (inf_v, big_v, inf_v, big_v))
    win_i[pl.ds(0, 16)] = bi0
    win_i[pl.ds(16, 16)] = bi1
    pltpu.async_copy(acc_hbm.at[win_i], vbuf, sem).wait()
    pltpu.sync_copy(vbuf, out_hbm.at[pl.ds(qbase, 32)])


def kernel(x, input_tensor, accuracy_tensor):
    keys = jnp.pad(input_tensor, (0, K_PAD - K_ORIG), constant_values=jnp.inf)
    pd, pi = _build_search(x, keys)
    return _merge_gather(pd, pi, accuracy_tensor)


# speedup vs baseline: 25.7195x; 1.8366x over previous
import functools

import jax
import jax.numpy as jnp
from jax import lax
from jax.experimental import pallas as pl
from jax.experimental.pallas import tpu as pltpu
from jax.experimental.pallas import tpu_sc as plsc

Q = 1024
K_ORIG = 100000
NT = 32
PER = 3136
K_PAD = NT * PER
NB = 8192
LOGB = 13
QV = Q // 16

_mesh = plsc.VectorSubcoreMesh(core_axis_name="c", subcore_axis_name="s")
_params = pltpu.CompilerParams(needs_layout_passes=False)


def _build_search_body(x_hbm, keys_hbm, outd_hbm, outi_hbm,
                       keys_v, xq_v, sv_v, si_v, off_v, ctr_v, claim_v,
                       bd_v, bi_v):
    wid = lax.axis_index("s") * 2 + lax.axis_index("c")
    base = wid * PER
    iota = lax.iota(jnp.int32, 16)
    ones_i = jnp.ones((16,), jnp.int32)

    pltpu.sync_copy(keys_hbm.at[pl.ds(base, PER)], keys_v)
    pltpu.sync_copy(x_hbm, xq_v)

    def mm_step(i, c):
        mn, mx = c
        k = keys_v[pl.ds(i * 16, 16)]
        mn = jnp.minimum(mn, k)
        mx = jnp.maximum(mx, jnp.where(k == jnp.inf, -jnp.inf, k))
        return mn, mx

    mn0 = jnp.full((16,), jnp.inf, jnp.float32)
    mx0 = jnp.full((16,), -jnp.inf, jnp.float32)
    mn, mx = lax.fori_loop(0, PER // 16, mm_step, (mn0, mx0))
    vmn = jnp.min(mn)
    vmx = jnp.max(mx)
    width = jnp.maximum(vmx - vmn, jnp.float32(1e-30))
    w16 = jnp.broadcast_to(width, (16,))
    ebits = (lax.bitcast_convert_type(w16, jnp.int32) >> 23) & 0xFF
    scale = lax.bitcast_convert_type((253 + LOGB - ebits) << 23, jnp.float32)
    invb = lax.bitcast_convert_type((ebits + 1 - LOGB) << 23, jnp.float32)

    def kbin(v):
        return jnp.clip((v - vmn) * scale, 0.0, float(NB - 1)).astype(jnp.int32)

    def z_step(i, _):
        ctr_v[pl.ds(i * 16, 16)] = jnp.zeros((16,), jnp.int32)
        return 0

    lax.fori_loop(0, NB // 16, z_step, 0)

    def h_step(i, _):
        k = keys_v[pl.ds(i * 16, 16)]
        plsc.addupdate_scatter(ctr_v, [kbin(k)], ones_i)
        return 0

    lax.fori_loop(0, PER // 16, h_step, 0)

    def s_step(i, carry):
        cnt = ctr_v[pl.ds(i * 16, 16)]
        cs = plsc.cumsum(cnt)
        excl = cs - cnt + carry
        off_v[pl.ds(i * 16, 16)] = excl
        ctr_v[pl.ds(i * 16, 16)] = excl
        return carry + jnp.sum(cnt)

    lax.fori_loop(0, NB // 16, s_step, jnp.int32(0))

    def p_step(i, _):
        k = keys_v[pl.ds(i * 16, 16)]
        g = base + i * 16 + iota
        b = kbin(k)

        def cond(c):
            return jnp.any(c[0])

        def body(c):
            act = c[0]
            plsc.store_scatter(claim_v, [b], iota, mask=act)
            win = (plsc.load_gather(claim_v, [b]) == iota) & act
            pos = plsc.load_gather(ctr_v, [b])
            plsc.store_scatter(sv_v, [pos], k, mask=win)
            plsc.store_scatter(si_v, [pos], g, mask=win)
            plsc.addupdate_scatter(ctr_v, [b], ones_i, mask=win)
            return (act & jnp.logical_not(win),)

        lax.while_loop(cond, body, (jnp.ones((16,), jnp.bool_),))
        return 0

    lax.fori_loop(0, PER // 16, p_step, 0)

    def q_step(j, _):
        q = xq_v[pl.ds(j * 16, 16)]
        cq = kbin(q)
        p0 = plsc.load_gather(off_v, [cq])
        bd0 = jnp.full((16,), jnp.inf, jnp.float32)
        bi0 = jnp.full((16,), 1 << 30, jnp.int32)

        def upd_best(act, v, gi, bd, bi):
            d = jnp.abs(q - v)
            u = act & ((d < bd) | ((d == bd) & (gi < bi)))
            return jnp.where(u, d, bd), jnp.where(u, gi, bi)

        def bi_cond(c):
            return jnp.any(c[0] | c[2])

        def bi_body(c):
            actu, curu, actd, curd, bd, bi = c
            ccu = jnp.clip(curu, 0, PER - 1)
            vu = plsc.load_gather(sv_v, [ccu])
            gu = plsc.load_gather(si_v, [ccu])
            ccd = jnp.clip(curd, 0, PER - 1)
            vd = plsc.load_gather(sv_v, [ccd])
            gd = plsc.load_gather(si_v, [ccd])
            bd, bi = upd_best(actu, vu, gu, bd, bi)
            bd, bi = upd_best(actd, vd, gd, bd, bi)
            lowb = vmn + (kbin(vu) - 1).astype(jnp.float32) * invb
            upb = vmn + (kbin(vd) + 2).astype(jnp.float32) * invb
            curu = curu + 1
            curd = curd - 1
            actu = actu & jnp.logical_not((lowb - q) > bd) & (curu < PER)
            actd = actd & jnp.logical_not((q - upb) > bd) & (curd >= 0)
            return actu, curu, actd, curd, bd, bi

        actu0 = p0 < PER
        actd0 = (p0 - 1) >= 0
        _, _, _, _, bd, bi = lax.while_loop(
            bi_cond, bi_body, (actu0, p0, actd0, p0 - 1, bd0, bi0))

        bd_v[pl.ds(j * 16, 16)] = bd
        bi_v[pl.ds(j * 16, 16)] = bi
        return 0

    lax.fori_loop(0, QV, q_step, 0)

    pltpu.sync_copy(bd_v, outd_hbm.at[wid])
    pltpu.sync_copy(bi_v, outi_hbm.at[wid])


@functools.partial(
    pl.kernel, mesh=_mesh, compiler_params=_params,
    out_type=[jax.ShapeDtypeStruct((NT, Q), jnp.float32),
              jax.ShapeDtypeStruct((NT, Q), jnp.int32)],
    scratch_types=[
        pltpu.VMEM((PER,), jnp.float32),
        pltpu.VMEM((Q,), jnp.float32),
        pltpu.VMEM((PER,), jnp.float32),
        pltpu.VMEM((PER,), jnp.int32),
        pltpu.VMEM((NB,), jnp.int32),
        pltpu.VMEM((NB,), jnp.int32),
        pltpu.VMEM((NB,), jnp.int32),
        pltpu.VMEM((Q,), jnp.float32),
        pltpu.VMEM((Q,), jnp.int32),
    ],
)
def _build_search(x_hbm, keys_hbm, outd_hbm, outi_hbm,
                  keys_v, xq_v, sv_v, si_v, off_v, ctr_v, claim_v, bd_v, bi_v):
    _build_search_body(x_hbm, keys_hbm, outd_hbm, outi_hbm,
                       keys_v, xq_v, sv_v, si_v, off_v, ctr_v, claim_v,
                       bd_v, bi_v)


@functools.partial(
    pl.kernel, mesh=_mesh, compiler_params=_params,
    out_type=jax.ShapeDtypeStruct((Q,), jnp.float32),
    scratch_types=[
        pltpu.VMEM((NT, 32), jnp.float32),
        pltpu.VMEM((NT, 32), jnp.int32),
        pltpu.VMEM((32,), jnp.int32),
        pltpu.VMEM((32,), jnp.float32),
        pltpu.SemaphoreType.DMA,
    ],
)
def _merge_gather(d_hbm, i_hbm, acc_hbm, out_hbm, dbuf, ibuf, win_i, vbuf, sem):
    wid = lax.axis_index("s") * 2 + lax.axis_index("c")
    qbase = wid * 32

    copies = []
    for t in range(NT):
        copies.append(pltpu.async_copy(
            d_hbm.at[t, pl.ds(qbase, 32)], dbuf.at[t], sem))
        copies.append(pltpu.async_copy(
            i_hbm.at[t, pl.ds(qbase, 32)], ibuf.at[t], sem))
    for c in copies:
        c.wait()

    def mix(bd, bi, d, gi):
        u = (d < bd) | ((d == bd) & (gi < bi))
        return jnp.where(u, d, bd), jnp.where(u, gi, bi)

    bd0 = jnp.full((16,), jnp.inf, jnp.float32)
    bd1 = bd0
    bi0 = jnp.full((16,), 1 << 30, jnp.int32)
    bi1 = bi0
    for t in range(NT):
        bd0, bi0 = mix(bd0, bi0, dbuf[t, pl.ds(0, 16)], ibuf[t, pl.ds(0, 16)])
        bd1, bi1 = mix(bd1, bi1, dbuf[t, pl.ds(16, 16)], ibuf[t, pl.ds(16, 16)])
    win_i[pl.ds(0, 16)] = bi0
    win_i[pl.ds(16, 16)] = bi1
    pltpu.async_copy(acc_hbm.at[win_i], vbuf, sem).wait()
    pltpu.sync_copy(vbuf, out_hbm.at[pl.ds(qbase, 32)])


def kernel(x, input_tensor, accuracy_tensor):
    keys = jnp.pad(input_tensor, (0, K_PAD - K_ORIG), constant_values=jnp.inf)
    pd, pi = _build_search(x, keys)
    return _merge_gather(pd, pi, accuracy_tensor)


# speedup vs baseline: 28.6785x; 1.1151x over previous
import functools

import jax
import jax.numpy as jnp
from jax import lax
from jax.experimental import pallas as pl
from jax.experimental.pallas import tpu as pltpu
from jax.experimental.pallas import tpu_sc as plsc

Q = 1024
K_ORIG = 100000
NT = 32
PER = 3136
K_PAD = NT * PER
NB = 8192
LOGB = 13
QV = Q // 16

_mesh = plsc.VectorSubcoreMesh(core_axis_name="c", subcore_axis_name="s")
_params = pltpu.CompilerParams(needs_layout_passes=False)


def _build_search_body(x_hbm, keys_hbm, outd_hbm, outi_hbm,
                       keys_v, xq_v, sv_v, si_v, off_v, ctr_v, claim_v,
                       bd_v, bi_v):
    wid = lax.axis_index("s") * 2 + lax.axis_index("c")
    base = wid * PER
    iota = lax.iota(jnp.int32, 16)
    ones_i = jnp.ones((16,), jnp.int32)

    nk = jnp.where(wid == NT - 1, K_ORIG - (NT - 1) * PER, PER)
    nv = nk // 16

    @pl.when(wid == NT - 1)
    def _():
        pltpu.sync_copy(keys_hbm.at[pl.ds(base, K_ORIG - (NT - 1) * PER)],
                        keys_v.at[pl.ds(0, K_ORIG - (NT - 1) * PER)])

    @pl.when(wid != NT - 1)
    def _():
        pltpu.sync_copy(keys_hbm.at[pl.ds(base, PER)], keys_v)

    pltpu.sync_copy(x_hbm, xq_v)

    def mm_step(i, c):
        mn, mx = c
        k = keys_v[pl.ds(i * 16, 16)]
        mn = jnp.minimum(mn, k)
        mx = jnp.maximum(mx, k)
        return mn, mx

    mn0 = jnp.full((16,), jnp.inf, jnp.float32)
    mx0 = jnp.full((16,), -jnp.inf, jnp.float32)
    mn, mx = lax.fori_loop(0, nv, mm_step, (mn0, mx0))
    vmn = jnp.min(mn)
    vmx = jnp.max(mx)
    width = jnp.maximum(vmx - vmn, jnp.float32(1e-30))
    w16 = jnp.broadcast_to(width, (16,))
    ebits = (lax.bitcast_convert_type(w16, jnp.int32) >> 23) & 0xFF
    scale = lax.bitcast_convert_type((253 + LOGB - ebits) << 23, jnp.float32)
    invb = lax.bitcast_convert_type((ebits + 1 - LOGB) << 23, jnp.float32)

    def kbin(v):
        return jnp.clip((v - vmn) * scale, 0.0, float(NB - 1)).astype(jnp.int32)

    def z_step(i, _):
        ctr_v[pl.ds(i * 16, 16)] = jnp.zeros((16,), jnp.int32)
        return 0

    lax.fori_loop(0, NB // 16, z_step, 0)

    def h_step(i, _):
        k = keys_v[pl.ds(i * 16, 16)]
        plsc.addupdate_scatter(ctr_v, [kbin(k)], ones_i)
        return 0

    lax.fori_loop(0, nv, h_step, 0)

    def s_step(i, carry):
        cnt = ctr_v[pl.ds(i * 16, 16)]
        cs = plsc.cumsum(cnt)
        excl = cs - cnt + carry
        off_v[pl.ds(i * 16, 16)] = excl
        ctr_v[pl.ds(i * 16, 16)] = excl
        return carry + jnp.sum(cnt)

    lax.fori_loop(0, NB // 16, s_step, jnp.int32(0))

    def p_step(i, _):
        k = keys_v[pl.ds(i * 16, 16)]
        g = base + i * 16 + iota
        b = kbin(k)

        def cond(c):
            return jnp.any(c[0])

        def body(c):
            act = c[0]
            plsc.store_scatter(claim_v, [b], iota, mask=act)
            win = (plsc.load_gather(claim_v, [b]) == iota) & act
            pos = plsc.load_gather(ctr_v, [b])
            plsc.store_scatter(sv_v, [pos], k, mask=win)
            plsc.store_scatter(si_v, [pos], g, mask=win)
            plsc.addupdate_scatter(ctr_v, [b], ones_i, mask=win)
            return (act & jnp.logical_not(win),)

        lax.while_loop(cond, body, (jnp.ones((16,), jnp.bool_),))
        return 0

    lax.fori_loop(0, nv, p_step, 0)

    def q_step(j, _):
        q = xq_v[pl.ds(j * 16, 16)]
        cq = kbin(q)
        p0 = plsc.load_gather(off_v, [cq])
        bd0 = jnp.full((16,), jnp.inf, jnp.float32)
        bi0 = jnp.full((16,), 1 << 30, jnp.int32)

        def upd_best(act, v, gi, bd, bi):
            d = jnp.abs(q - v)
            u = act & ((d < bd) | ((d == bd) & (gi < bi)))
            return jnp.where(u, d, bd), jnp.where(u, gi, bi)

        def bi_cond(c):
            return jnp.any(c[0] | c[2])

        def bi_body(c):
            actu, curu, actd, curd, bd, bi = c
            ccu = jnp.clip(curu, 0, PER - 1)
            vu = plsc.load_gather(sv_v, [ccu])
            gu = plsc.load_gather(si_v, [ccu])
            ccd = jnp.clip(curd, 0, PER - 1)
            vd = plsc.load_gather(sv_v, [ccd])
            gd = plsc.load_gather(si_v, [ccd])
            bd, bi = upd_best(actu, vu, gu, bd, bi)
            bd, bi = upd_best(actd, vd, gd, bd, bi)
            lowb = vmn + (kbin(vu).astype(jnp.float32) - 0.02) * invb
            upb = vmn + (kbin(vd).astype(jnp.float32) + 1.02) * invb
            curu = curu + 1
            curd = curd - 1
            actu = actu & jnp.logical_not((lowb - q) > bd) & (curu < nk)
            actd = actd & jnp.logical_not((q - upb) > bd) & (curd >= 0)
            return actu, curu, actd, curd, bd, bi

        actu0 = p0 < nk
        actd0 = (p0 - 1) >= 0
        _, _, _, _, bd, bi = lax.while_loop(
            bi_cond, bi_body, (actu0, p0, actd0, p0 - 1, bd0, bi0))

        bd_v[pl.ds(j * 16, 16)] = bd
        bi_v[pl.ds(j * 16, 16)] = bi
        return 0

    lax.fori_loop(0, QV, q_step, 0)

    pltpu.sync_copy(bd_v, outd_hbm.at[wid])
    pltpu.sync_copy(bi_v, outi_hbm.at[wid])


@functools.partial(
    pl.kernel, mesh=_mesh, compiler_params=_params,
    out_type=[jax.ShapeDtypeStruct((NT, Q), jnp.float32),
              jax.ShapeDtypeStruct((NT, Q), jnp.int32)],
    scratch_types=[
        pltpu.VMEM((PER,), jnp.float32),
        pltpu.VMEM((Q,), jnp.float32),
        pltpu.VMEM((PER,), jnp.float32),
        pltpu.VMEM((PER,), jnp.int32),
        pltpu.VMEM((NB,), jnp.int32),
        pltpu.VMEM((NB,), jnp.int32),
        pltpu.VMEM((NB,), jnp.int32),
        pltpu.VMEM((Q,), jnp.float32),
        pltpu.VMEM((Q,), jnp.int32),
    ],
)
def _build_search(x_hbm, keys_hbm, outd_hbm, outi_hbm,
                  keys_v, xq_v, sv_v, si_v, off_v, ctr_v, claim_v, bd_v, bi_v):
    _build_search_body(x_hbm, keys_hbm, outd_hbm, outi_hbm,
                       keys_v, xq_v, sv_v, si_v, off_v, ctr_v, claim_v,
                       bd_v, bi_v)


@functools.partial(
    pl.kernel, mesh=_mesh, compiler_params=_params,
    out_type=jax.ShapeDtypeStruct((Q,), jnp.float32),
    scratch_types=[
        pltpu.VMEM((NT, 32), jnp.float32),
        pltpu.VMEM((NT, 32), jnp.int32),
        pltpu.VMEM((32,), jnp.int32),
        pltpu.VMEM((32,), jnp.float32),
        pltpu.SemaphoreType.DMA,
    ],
)
def _merge_gather(d_hbm, i_hbm, acc_hbm, out_hbm, dbuf, ibuf, win_i, vbuf, sem):
    wid = lax.axis_index("s") * 2 + lax.axis_index("c")
    qbase = wid * 32

    copies = []
    for t in range(NT):
        copies.append(pltpu.async_copy(
            d_hbm.at[t, pl.ds(qbase, 32)], dbuf.at[t], sem))
        copies.append(pltpu.async_copy(
            i_hbm.at[t, pl.ds(qbase, 32)], ibuf.at[t], sem))
    for c in copies:
        c.wait()

    def mix(bd, bi, d, gi):
        u = (d < bd) | ((d == bd) & (gi < bi))
        return jnp.where(u, d, bd), jnp.where(u, gi, bi)

    bd0 = jnp.full((16,), jnp.inf, jnp.float32)
    bd1 = bd0
    bi0 = jnp.full((16,), 1 << 30, jnp.int32)
    bi1 = bi0
    for t in range(NT):
        bd0, bi0 = mix(bd0, bi0, dbuf[t, pl.ds(0, 16)], ibuf[t, pl.ds(0, 16)])
        bd1, bi1 = mix(bd1, bi1, dbuf[t, pl.ds(16, 16)], ibuf[t, pl.ds(16, 16)])
    win_i[pl.ds(0, 16)] = bi0
    win_i[pl.ds(16, 16)] = bi1
    pltpu.async_copy(acc_hbm.at[win_i], vbuf, sem).wait()
    pltpu.sync_copy(vbuf, out_hbm.at[pl.ds(qbase, 32)])


def kernel(x, input_tensor, accuracy_tensor):
    pd, pi = _build_search(x, input_tensor)
    return _merge_gather(pd, pi, accuracy_tensor)
